# baseline (speedup 1.0000x reference)
import functools

import jax
import jax.numpy as jnp
from jax import lax
from jax.experimental import pallas as pl
from jax.experimental.pallas import tpu as pltpu
from jax.experimental.pallas import tpu_sc as plsc

E = 8
IN = 2048
MID = 8192
OUT = 2048
S = 4096
C = S // E
BS = 512
NB = S // BS
TRASH = 512
BM1 = 2048
NM1 = MID // BM1
BM2 = 512
NS2 = BM1 // BM2

NC, NS = 2, 16
NW = NC * NS
RPW = S // NW
CH = 16
NCH = RPW // CH


def _gate_body(x_ref, wg_ref, slot_ref, w16_ref, laux_ref,
               cnt_ref, me_ref, ce_ref):
    i = pl.program_id(0)

    @pl.when(i == 0)
    def _init():
        cnt_ref[...] = jnp.zeros_like(cnt_ref)
        me_ref[...] = jnp.zeros_like(me_ref)
        ce_ref[...] = jnp.zeros_like(ce_ref)

    x = x_ref[...]
    logits = jnp.dot(x, wg_ref[...],
                     preferred_element_type=jnp.float32)
    lmax = jnp.max(logits, axis=1, keepdims=True)
    p = jnp.exp(logits - lmax)
    gates = p / jnp.sum(p, axis=1, keepdims=True)

    iota_e = lax.broadcasted_iota(jnp.int32, gates.shape, 1)
    gmax = jnp.max(gates, axis=1, keepdims=True)
    eidx = jnp.min(jnp.where(gates == gmax, iota_e, E), axis=1, keepdims=True)
    mask = (iota_e == eidx).astype(jnp.float32)

    me_ref[...] += jnp.sum(gates, axis=0, keepdims=True)
    ce_ref[...] += jnp.sum(mask, axis=0, keepdims=True)

    r = lax.broadcasted_iota(jnp.int32, (BS, BS), 0)
    c = lax.broadcasted_iota(jnp.int32, (BS, BS), 1)
    tri = (r >= c).astype(jnp.float32)
    incl = lax.dot(tri, mask, precision=lax.Precision.HIGHEST) + cnt_ref[...]
    cnt_ref[...] += jnp.sum(mask, axis=0, keepdims=True)
    loc = incl - 1.0

    maskk = mask * (loc < C).astype(jnp.float32)
    pos = jnp.sum(loc * maskk, axis=1, keepdims=True)
    g_s = jnp.sum(gates * maskk, axis=1, keepdims=True)
    kept = jnp.sum(maskk, axis=1, keepdims=True) > 0.0

    dflat = eidx * C + pos.astype(jnp.int32)
    slot_ref[...] = jnp.where(kept, dflat, S)
    w16_ref[...] = jnp.where(kept, g_s, 0.0) * jnp.ones((1, 128), jnp.float32)

    @pl.when(i == NB - 1)
    def _fin():
        me = me_ref[...] / float(S)
        ce = ce_ref[...] / float(S)
        laux_ref[...] = jnp.sum(me * ce, axis=1, keepdims=True) * float(E)


def _gate(feats, wg):
    return pl.pallas_call(
        _gate_body,
        grid=(NB,),
        in_specs=[
            pl.BlockSpec((BS, IN), lambda i: (i, 0)),
            pl.BlockSpec((IN, E), lambda i: (0, 0)),
        ],
        out_specs=[
            pl.BlockSpec((BS, 1), lambda i: (i, 0)),
            pl.BlockSpec((BS, 128), lambda i: (i, 0)),
            pl.BlockSpec((1, 1), lambda i: (0, 0)),
        ],
        out_shape=[
            jax.ShapeDtypeStruct((S, 1), jnp.int32),
            jax.ShapeDtypeStruct((S, 128), jnp.float32),
            jax.ShapeDtypeStruct((1, 1), jnp.float32),
        ],
        scratch_shapes=[
            pltpu.VMEM((1, E), jnp.float32),
            pltpu.VMEM((1, E), jnp.float32),
            pltpu.VMEM((1, E), jnp.float32),
        ],
    )(feats, wg)


@functools.cache
def _sc_kernels():
    mesh = plsc.VectorSubcoreMesh(core_axis_name="c", subcore_axis_name="s")

    @functools.partial(
        pl.kernel,
        mesh=mesh,
        out_type=[
            jax.ShapeDtypeStruct((S + TRASH, IN), jnp.float32),
            jax.ShapeDtypeStruct((S + TRASH, 128), jnp.float32),
        ],
        scratch_types=[
            pltpu.VMEM((NCH, CH), jnp.int32),
            pltpu.VMEM((CH, IN), jnp.float32),
            pltpu.VMEM((CH, IN), jnp.float32),
            pltpu.VMEM((RPW, 128), jnp.float32),
            pltpu.SemaphoreType.DMA,
            pltpu.SemaphoreType.DMA,
            pltpu.SemaphoreType.DMA,
            pltpu.SemaphoreType.DMA,
            pltpu.SemaphoreType.DMA,
        ],
    )
    def dispatch_sc(feat_hbm, w16_hbm, idx_hbm, out_hbm, wdest_hbm,
                    idx_v, b0, b1, w_v, sl0, sl1, ss0, ss1, sw):
        wid = lax.axis_index("s") * NC + lax.axis_index("c")
        base = wid * RPW
        bufs, sls, sss = (b0, b1), (sl0, sl1), (ss0, ss1)
        pltpu.sync_copy(idx_hbm.at[wid], idx_v)
        pltpu.sync_copy(w16_hbm.at[pl.ds(base, RPW)], w_v)
        wst = [None] * NCH
        loads = [None] * NCH
        stores = [None] * NCH
        loads[0] = pltpu.async_copy(feat_hbm.at[pl.ds(base, CH)], bufs[0], sls[0])
        for j in range(NCH):
            k = j % 2
            loads[j].wait()
            if j + 1 < NCH:
                if j >= 1:
                    stores[j - 1].wait()
                loads[j + 1] = pltpu.async_copy(
                    feat_hbm.at[pl.ds(base + (j + 1) * CH, CH)],
                    bufs[(j + 1) % 2], sls[(j + 1) % 2])
            stores[j] = pltpu.async_copy(bufs[k], out_hbm.at[idx_v.at[j]], sss[k])
            wst[j] = pltpu.async_copy(
                w_v.at[pl.ds(j * CH, CH)], wdest_hbm.at[idx_v.at[j]], sw)
        stores[NCH - 2].wait()
        stores[NCH - 1].wait()
        for j in range(NCH):
            wst[j].wait()

    @functools.partial(
        pl.kernel,
        mesh=mesh,
        out_type=jax.ShapeDtypeStruct((S, OUT), jnp.float32),
        scratch_types=[
            pltpu.VMEM((NCH, CH), jnp.int32),
            pltpu.VMEM((CH, OUT), jnp.float32),
            pltpu.VMEM((CH, OUT), jnp.float32),
            pltpu.SemaphoreType.DMA,
            pltpu.SemaphoreType.DMA,
            pltpu.SemaphoreType.DMA,
            pltpu.SemaphoreType.DMA,
        ],
    )
    def combine_sc(eo_hbm, idx_hbm, out_hbm, idx_v, b0, b1, sl0, sl1,
                   ss0, ss1):
        wid = lax.axis_index("s") * NC + lax.axis_index("c")
        base = wid * RPW
        bufs, sls, sss = (b0, b1), (sl0, sl1), (ss0, ss1)
        pltpu.sync_copy(idx_hbm.at[wid], idx_v)
        loads = [None] * NCH
        stores = [None] * NCH
        loads[0] = pltpu.async_copy(eo_hbm.at[idx_v.at[0]], bufs[0], sls[0])
        for j in range(NCH):
            k = j % 2
            loads[j].wait()
            if j + 1 < NCH:
                if j >= 1:
                    stores[j - 1].wait()
                loads[j + 1] = pltpu.async_copy(
                    eo_hbm.at[idx_v.at[j + 1]], bufs[(j + 1) % 2],
                    sls[(j + 1) % 2])
            stores[j] = pltpu.async_copy(
                bufs[k], out_hbm.at[pl.ds(base + j * CH, CH)], sss[k])
        stores[NCH - 2].wait()
        stores[NCH - 1].wait()

    return dispatch_sc, combine_sc


def _ffn_body(x_ref, w1_ref, b1_ref, w2_ref, b2_ref, wd_ref, o_ref):
    e = pl.program_id(0)
    m1 = pl.program_id(1)
    m2 = pl.program_id(2)

    @pl.when(e < E)
    def _compute():
        x = x_ref[...].astype(jnp.bfloat16)
        w1p = w1_ref[0, :, pl.ds(m2 * BM2, BM2)].astype(jnp.bfloat16)
        h = jnp.dot(x, w1p, preferred_element_type=jnp.float32)
        h = jnp.maximum(h + b1_ref[0, 0, pl.ds(m2 * BM2, BM2)], 0.0)
        w2 = w2_ref[0].astype(jnp.bfloat16)
        contrib = jnp.dot(h.astype(jnp.bfloat16), w2,
                          preferred_element_type=jnp.float32)
        first = (m1 == 0) & (m2 == 0)
        last = (m1 == NM1 - 1) & (m2 == NS2 - 1)

        @pl.when(first)
        def _first():
            o_ref[...] = contrib + b2_ref[0]

        @pl.when(jnp.logical_not(first | last))
        def _mid():
            o_ref[...] += contrib

        @pl.when(last)
        def _last():
            o_ref[...] = (o_ref[...] + contrib) * wd_ref[...][:, :1]

    @pl.when((e == E) & (m1 == 0) & (m2 == 0))
    def _zero_block():
        o_ref[...] = jnp.zeros_like(o_ref)


def _ffn(disp, W1, b1, W2, b2, wdest):
    return pl.pallas_call(
        _ffn_body,
        grid=(E + 1, NM1, NS2),
        in_specs=[
            pl.BlockSpec((C, IN),
                         lambda e, m1, m2: (jnp.minimum(e, E - 1), 0)),
            pl.BlockSpec((1, IN, BM1),
                         lambda e, m1, m2: (jnp.minimum(e, E - 1),
                                            0,
                                            jnp.where(e == E, NM1 - 1, m1))),
            pl.BlockSpec((1, 1, BM1),
                         lambda e, m1, m2: (jnp.minimum(e, E - 1),
                                            0,
                                            jnp.where(e == E, NM1 - 1, m1))),
            pl.BlockSpec((1, BM2, OUT),
                         lambda e, m1, m2: (jnp.minimum(e, E - 1),
                                            jnp.where(e == E,
                                                      NM1 * NS2 - 1,
                                                      m1 * NS2 + m2),
                                            0)),
            pl.BlockSpec((1, 1, OUT),
                         lambda e, m1, m2: (jnp.minimum(e, E - 1), 0, 0)),
            pl.BlockSpec((C, 128), lambda e, m1, m2: (e, 0)),
        ],
        out_specs=pl.BlockSpec((C, OUT), lambda e, m1, m2: (e, 0)),
        out_shape=jax.ShapeDtypeStruct((S + TRASH, OUT), jnp.float32),
        compiler_params=pltpu.CompilerParams(
            vmem_limit_bytes=100 * 1024 * 1024),
    )(disp, W1, b1, W2, b2, wdest)


def kernel(hidden_states, wg, W1, b1, W2, b2):
    B, T, M = hidden_states.shape
    feats = hidden_states.reshape(S, M)

    slot, w16, laux = _gate(feats, wg)
    idx3 = slot.reshape(NW, NCH, CH)

    dispatch_sc, combine_sc = _sc_kernels()
    disp, wdest = dispatch_sc(feats, w16, idx3)
    eo = _ffn(disp, W1, b1.reshape(E, 1, MID), W2, b2.reshape(E, 1, OUT),
              wdest)
    out = combine_sc(eo, idx3)

    return out.reshape(B, T, OUT), laux[0, 0]


# speedup vs baseline: 1.3446x; 1.3446x over previous
import functools

import jax
import jax.numpy as jnp
from jax import lax
from jax.experimental import pallas as pl
from jax.experimental.pallas import tpu as pltpu
from jax.experimental.pallas import tpu_sc as plsc

E = 8
IN = 2048
MID = 8192
OUT = 2048
S = 4096
C = S // E
BS = 512
NB = S // BS
TRASH = 512
BM = 1024
NM = MID // BM

NC, NS = 2, 16
NW = NC * NS
RPW = S // NW
CH = 16
NCH = RPW // CH


def _gate_body(x_ref, wg_ref, slot_ref, w16_ref, laux_ref,
               cnt_ref, me_ref, ce_ref):
    i = pl.program_id(0)

    @pl.when(i == 0)
    def _init():
        cnt_ref[...] = jnp.zeros_like(cnt_ref)
        me_ref[...] = jnp.zeros_like(me_ref)
        ce_ref[...] = jnp.zeros_like(ce_ref)

    x = x_ref[...]
    logits = jnp.dot(x, wg_ref[...],
                     preferred_element_type=jnp.float32)
    lmax = jnp.max(logits, axis=1, keepdims=True)
    p = jnp.exp(logits - lmax)
    gates = p / jnp.sum(p, axis=1, keepdims=True)

    iota_e = lax.broadcasted_iota(jnp.int32, gates.shape, 1)
    gmax = jnp.max(gates, axis=1, keepdims=True)
    eidx = jnp.min(jnp.where(gates == gmax, iota_e, E), axis=1, keepdims=True)
    mask = (iota_e == eidx).astype(jnp.float32)

    me_ref[...] += jnp.sum(gates, axis=0, keepdims=True)
    ce_ref[...] += jnp.sum(mask, axis=0, keepdims=True)

    r = lax.broadcasted_iota(jnp.int32, (BS, BS), 0)
    c = lax.broadcasted_iota(jnp.int32, (BS, BS), 1)
    tri = (r >= c).astype(jnp.float32)
    incl = lax.dot(tri, mask, precision=lax.Precision.HIGHEST) + cnt_ref[...]
    cnt_ref[...] += jnp.sum(mask, axis=0, keepdims=True)
    loc = incl - 1.0

    maskk = mask * (loc < C).astype(jnp.float32)
    pos = jnp.sum(loc * maskk, axis=1, keepdims=True)
    g_s = jnp.sum(gates * maskk, axis=1, keepdims=True)
    kept = jnp.sum(maskk, axis=1, keepdims=True) > 0.0

    dflat = eidx * C + pos.astype(jnp.int32)
    slot_ref[...] = jnp.where(kept, dflat, S)
    w16_ref[...] = jnp.where(kept, g_s, 0.0) * jnp.ones((1, 128), jnp.float32)

    @pl.when(i == NB - 1)
    def _fin():
        me = me_ref[...] / float(S)
        ce = ce_ref[...] / float(S)
        laux_ref[...] = jnp.sum(me * ce, axis=1, keepdims=True) * float(E)


def _gate(feats, wg):
    return pl.pallas_call(
        _gate_body,
        grid=(NB,),
        in_specs=[
            pl.BlockSpec((BS, IN), lambda i: (i, 0)),
            pl.BlockSpec((IN, E), lambda i: (0, 0)),
        ],
        out_specs=[
            pl.BlockSpec((BS, 1), lambda i: (i, 0)),
            pl.BlockSpec((BS, 128), lambda i: (i, 0)),
            pl.BlockSpec((1, 1), lambda i: (0, 0)),
        ],
        out_shape=[
            jax.ShapeDtypeStruct((S, 1), jnp.int32),
            jax.ShapeDtypeStruct((S, 128), jnp.float32),
            jax.ShapeDtypeStruct((1, 1), jnp.float32),
        ],
        scratch_shapes=[
            pltpu.VMEM((1, E), jnp.float32),
            pltpu.VMEM((1, E), jnp.float32),
            pltpu.VMEM((1, E), jnp.float32),
        ],
    )(feats, wg)


@functools.cache
def _sc_kernels():
    mesh = plsc.VectorSubcoreMesh(core_axis_name="c", subcore_axis_name="s")

    @functools.partial(
        pl.kernel,
        mesh=mesh,
        out_type=[
            jax.ShapeDtypeStruct((S + TRASH, IN), jnp.float32),
            jax.ShapeDtypeStruct((S + TRASH, 128), jnp.float32),
        ],
        scratch_types=[
            pltpu.VMEM((NCH, CH), jnp.int32),
            pltpu.VMEM((CH, IN), jnp.float32),
            pltpu.VMEM((CH, IN), jnp.float32),
            pltpu.VMEM((RPW, 128), jnp.float32),
            pltpu.SemaphoreType.DMA,
            pltpu.SemaphoreType.DMA,
            pltpu.SemaphoreType.DMA,
            pltpu.SemaphoreType.DMA,
            pltpu.SemaphoreType.DMA,
        ],
    )
    def dispatch_sc(feat_hbm, w16_hbm, idx_hbm, out_hbm, wdest_hbm,
                    idx_v, b0, b1, w_v, sl0, sl1, ss0, ss1, sw):
        wid = lax.axis_index("s") * NC + lax.axis_index("c")
        base = wid * RPW
        bufs, sls, sss = (b0, b1), (sl0, sl1), (ss0, ss1)
        pltpu.sync_copy(idx_hbm.at[wid], idx_v)
        pltpu.sync_copy(w16_hbm.at[pl.ds(base, RPW)], w_v)
        wst = [None] * NCH
        loads = [None] * NCH
        stores = [None] * NCH
        loads[0] = pltpu.async_copy(feat_hbm.at[pl.ds(base, CH)], bufs[0], sls[0])
        for j in range(NCH):
            k = j % 2
            loads[j].wait()
            if j + 1 < NCH:
                if j >= 1:
                    stores[j - 1].wait()
                loads[j + 1] = pltpu.async_copy(
                    feat_hbm.at[pl.ds(base + (j + 1) * CH, CH)],
                    bufs[(j + 1) % 2], sls[(j + 1) % 2])
            stores[j] = pltpu.async_copy(bufs[k], out_hbm.at[idx_v.at[j]], sss[k])
            wst[j] = pltpu.async_copy(
                w_v.at[pl.ds(j * CH, CH)], wdest_hbm.at[idx_v.at[j]], sw)
        stores[NCH - 2].wait()
        stores[NCH - 1].wait()
        for j in range(NCH):
            wst[j].wait()

    @functools.partial(
        pl.kernel,
        mesh=mesh,
        out_type=jax.ShapeDtypeStruct((S, OUT), jnp.float32),
        scratch_types=[
            pltpu.VMEM((NCH, CH), jnp.int32),
            pltpu.VMEM((CH, OUT), jnp.float32),
            pltpu.VMEM((CH, OUT), jnp.float32),
            pltpu.SemaphoreType.DMA,
            pltpu.SemaphoreType.DMA,
            pltpu.SemaphoreType.DMA,
            pltpu.SemaphoreType.DMA,
        ],
    )
    def combine_sc(eo_hbm, idx_hbm, out_hbm, idx_v, b0, b1, sl0, sl1,
                   ss0, ss1):
        wid = lax.axis_index("s") * NC + lax.axis_index("c")
        base = wid * RPW
        bufs, sls, sss = (b0, b1), (sl0, sl1), (ss0, ss1)
        pltpu.sync_copy(idx_hbm.at[wid], idx_v)
        loads = [None] * NCH
        stores = [None] * NCH
        loads[0] = pltpu.async_copy(eo_hbm.at[idx_v.at[0]], bufs[0], sls[0])
        for j in range(NCH):
            k = j % 2
            loads[j].wait()
            if j + 1 < NCH:
                if j >= 1:
                    stores[j - 1].wait()
                loads[j + 1] = pltpu.async_copy(
                    eo_hbm.at[idx_v.at[j + 1]], bufs[(j + 1) % 2],
                    sls[(j + 1) % 2])
            stores[j] = pltpu.async_copy(
                bufs[k], out_hbm.at[pl.ds(base + j * CH, CH)], sss[k])
        stores[NCH - 2].wait()
        stores[NCH - 1].wait()

    return dispatch_sc, combine_sc


def _ffn_body(x_ref, w1_ref, b1_ref, w2_ref, b2_ref, wd_ref, o_ref, xb_ref):
    e = pl.program_id(0)
    m = pl.program_id(1)

    @pl.when(e < E)
    def _compute():
        @pl.when(m == 0)
        def _cast_x():
            xb_ref[...] = x_ref[...].astype(jnp.bfloat16)

        w1 = w1_ref[0].astype(jnp.bfloat16)
        h = jnp.dot(xb_ref[...], w1, preferred_element_type=jnp.float32)
        h = jnp.maximum(h + b1_ref[0], 0.0)
        w2 = w2_ref[0].astype(jnp.bfloat16)
        contrib = jnp.dot(h.astype(jnp.bfloat16), w2,
                          preferred_element_type=jnp.float32)

        @pl.when(m == 0)
        def _first():
            o_ref[...] = contrib + b2_ref[0]

        @pl.when((m > 0) & (m < NM - 1))
        def _mid():
            o_ref[...] += contrib

        @pl.when(m == NM - 1)
        def _last():
            o_ref[...] = (o_ref[...] + contrib) * wd_ref[...][:, :1]

    @pl.when((e == E) & (m == 0))
    def _zero_block():
        o_ref[...] = jnp.zeros_like(o_ref)


def _ffn(disp, W1, b1, W2, b2, wdest):
    return pl.pallas_call(
        _ffn_body,
        grid=(E + 1, NM),
        in_specs=[
            pl.BlockSpec((C, IN), lambda e, m: (jnp.minimum(e, E - 1), 0)),
            pl.BlockSpec((1, IN, BM),
                         lambda e, m: (jnp.minimum(e, E - 1),
                                       0,
                                       jnp.where(e == E, NM - 1, m))),
            pl.BlockSpec((1, 1, BM),
                         lambda e, m: (jnp.minimum(e, E - 1),
                                       0,
                                       jnp.where(e == E, NM - 1, m))),
            pl.BlockSpec((1, BM, OUT),
                         lambda e, m: (jnp.minimum(e, E - 1),
                                       jnp.where(e == E, NM - 1, m),
                                       0)),
            pl.BlockSpec((1, 1, OUT), lambda e, m: (jnp.minimum(e, E - 1), 0, 0)),
            pl.BlockSpec((C, 128), lambda e, m: (e, 0)),
        ],
        out_specs=pl.BlockSpec((C, OUT), lambda e, m: (e, 0)),
        out_shape=jax.ShapeDtypeStruct((S + TRASH, OUT), jnp.float32),
        scratch_shapes=[pltpu.VMEM((C, IN), jnp.bfloat16)],
        compiler_params=pltpu.CompilerParams(
            vmem_limit_bytes=100 * 1024 * 1024),
    )(disp, W1, b1, W2, b2, wdest)


def kernel(hidden_states, wg, W1, b1, W2, b2):
    B, T, M = hidden_states.shape
    feats = hidden_states.reshape(S, M)

    slot, w16, laux = _gate(feats, wg)
    idx3 = slot.reshape(NW, NCH, CH)

    dispatch_sc, combine_sc = _sc_kernels()
    disp, wdest = dispatch_sc(feats, w16, idx3)
    eo = _ffn(disp, W1, b1.reshape(E, 1, MID), W2, b2.reshape(E, 1, OUT),
              wdest)
    out = combine_sc(eo, idx3)

    return out.reshape(B, T, OUT), laux[0, 0]
